# baseline (speedup 1.0000x reference)
import jax
import jax.numpy as jnp
from jax.experimental import pallas as pl

DIM = 32
HEADS = 4
HID = DIM * HEADS
N_DST = 5000
NEG_SLOPE = 0.1


def _touch_kernel(x_ref, o_ref):
    o_ref[...] = x_ref[...]


def kernel(n_id, edge_src, edge_dst, weights, size0, size1, pre_w, pre_b,
           lin_src_w, lin_dst_w, att_src, att_dst, gat_bias):
    s0 = n_id.shape[0]
    s1 = N_DST
    x = jnp.take(pre_w.T, n_id, axis=0) + pre_b
    x_dst_in = x[:s1]
    loop = jnp.arange(s1)
    src = jnp.concatenate([edge_src, loop])
    dst = jnp.concatenate([edge_dst, loop])
    w = jnp.concatenate([weights, jnp.ones((s1,), jnp.float32)])
    xs = (x @ lin_src_w.T).reshape(s0, HEADS, DIM)
    xd = (x_dst_in @ lin_dst_w.T).reshape(s1, HEADS, DIM)
    a_src = (xs * att_src[None, :, :]).sum(-1)
    a_dst = (xd * att_dst[None, :, :]).sum(-1)
    alpha = a_src[src] + a_dst[dst]
    alpha = jax.nn.leaky_relu(alpha, NEG_SLOPE)
    amax = jax.ops.segment_max(alpha, dst, num_segments=s1)
    ex = jnp.exp(alpha - amax[dst])
    denom = jax.ops.segment_sum(ex, dst, num_segments=s1)
    attn = ex / (denom[dst] + 1e-16)
    attn = attn * w[:, None]
    msg = xs[src] * attn[:, :, None]
    out = jax.ops.segment_sum(msg, dst, num_segments=s1)
    out = out.reshape(s1, HEADS * DIM) + gat_bias
    out = pl.pallas_call(
        _touch_kernel,
        out_shape=jax.ShapeDtypeStruct(out.shape, out.dtype),
    )(out)
    return out


# speedup vs baseline: 16.7648x; 16.7648x over previous
import functools

import jax
import jax.numpy as jnp
from jax import lax
from jax.experimental import pallas as pl
from jax.experimental.pallas import tpu as pltpu
from jax.experimental.pallas import tpu_sc as plsc

IN_SIZE = 100000
DIM = 32
HEADS = 4
HID = DIM * HEADS
N_SRC = 10000
NSP = 10240
N_DST = 5000
E = 320000
NEG_SLOPE = 0.1

NC = 2
NS = 16
NW = NC * NS
CH = 128
NCH = 80
EPW = CH * NCH
EPAD = EPW * NW
ACC_ROWS = 5120
RPT = ACC_ROWS // NS

_f32 = jnp.float32
_i32 = jnp.int32


def _xpose_body(pre_w_hbm, nid_hbm, out_hbm, rowbuf, nid_v, outv):
    cid = lax.axis_index("c")
    sid = lax.axis_index("s")
    wid = sid * NC + cid
    pltpu.sync_copy(nid_hbm, nid_v)
    nrow = HID // NW

    def do_row(q, _):
        r = wid * nrow + q
        pltpu.sync_copy(pre_w_hbm.at[r], rowbuf)

        def gth(j, _):
            idx = nid_v[pl.ds(j * 16, 16)]
            outv[pl.ds(j * 16, 16)] = plsc.load_gather(rowbuf, [idx])
            return 0

        lax.fori_loop(0, NSP // 16, gth, 0)
        pltpu.sync_copy(outv, out_hbm.at[r])
        return 0

    lax.fori_loop(0, nrow, do_row, 0)


def _sc_transpose_gather(pre_w, n_id):
    mesh = plsc.VectorSubcoreMesh(core_axis_name="c", subcore_axis_name="s")
    f = functools.partial(
        pl.kernel,
        mesh=mesh,
        compiler_params=pltpu.CompilerParams(needs_layout_passes=False),
        out_type=jax.ShapeDtypeStruct((HID, NSP), _f32),
        scratch_types=[
            pltpu.VMEM((IN_SIZE,), _f32),
            pltpu.VMEM((NSP,), _i32),
            pltpu.VMEM((NSP,), _f32),
        ],
    )(_xpose_body)
    return f(pre_w, n_id)


_BN = 1024


def _dense_body(xc_ref, ws_ref, wd_ref, pb_ref, ats_ref, atd_ref,
                xs_out, as_out, ad_out):
    xc = xc_ref[...]
    ws = ws_ref[...]
    wd = wd_ref[...]
    pb = pb_ref[...]
    dn = (((0,), (1,)), ((), ()))
    bias_s = lax.dot_general(pb, ws, dimension_numbers=(((1,), (1,)), ((), ())),
                             preferred_element_type=_f32)
    bias_d = lax.dot_general(pb, wd, dimension_numbers=(((1,), (1,)), ((), ())),
                             preferred_element_type=_f32)
    xs = lax.dot_general(xc, ws, dimension_numbers=dn,
                         preferred_element_type=_f32) + bias_s
    xs_out[...] = xs

    ts = ws * ats_ref[...]
    td = wd * atd_ref[...]
    vs = jnp.concatenate(
        [jnp.sum(ts[DIM * h:DIM * (h + 1), :], axis=0, keepdims=True)
         for h in range(HEADS)], axis=0)
    vd = jnp.concatenate(
        [jnp.sum(td[DIM * h:DIM * (h + 1), :], axis=0, keepdims=True)
         for h in range(HEADS)], axis=0)
    bs = lax.dot_general(vs, pb, dimension_numbers=(((1,), (1,)), ((), ())),
                         preferred_element_type=_f32)
    bd = lax.dot_general(vd, pb, dimension_numbers=(((1,), (1,)), ((), ())),
                         preferred_element_type=_f32)
    as_out[...] = lax.dot_general(vs, xc, dimension_numbers=(((1,), (0,)), ((), ())),
                                  preferred_element_type=_f32) + bs
    ad_out[...] = lax.dot_general(vd, xc, dimension_numbers=(((1,), (0,)), ((), ())),
                                  preferred_element_type=_f32) + bd


def _tc_dense(x_colT, lin_src_w, lin_dst_w, pre_b2, ats2, atd2):
    grid = (NSP // _BN,)
    return pl.pallas_call(
        _dense_body,
        grid=grid,
        in_specs=[
            pl.BlockSpec((HID, _BN), lambda i: (0, i)),
            pl.BlockSpec((HID, HID), lambda i: (0, 0)),
            pl.BlockSpec((HID, HID), lambda i: (0, 0)),
            pl.BlockSpec((1, HID), lambda i: (0, 0)),
            pl.BlockSpec((HID, 1), lambda i: (0, 0)),
            pl.BlockSpec((HID, 1), lambda i: (0, 0)),
        ],
        out_specs=[
            pl.BlockSpec((_BN, HID), lambda i: (i, 0)),
            pl.BlockSpec((HEADS, _BN), lambda i: (0, i)),
            pl.BlockSpec((HEADS, _BN), lambda i: (0, i)),
        ],
        out_shape=[
            jax.ShapeDtypeStruct((NSP, HID), _f32),
            jax.ShapeDtypeStruct((HEADS, NSP), _f32),
            jax.ShapeDtypeStruct((HEADS, NSP), _f32),
        ],
    )(x_colT, lin_src_w, lin_dst_w, pre_b2, ats2, atd2)


def _edge_body(src_hbm, dst_hbm, w_hbm, xs_hbm, as_hbm, ad_hbm, zn_hbm, zd_hbm,
               np_hbm, dp_hbm,
               asrc_v, adst_v, sv, dv, wv, G, pH, cT, ix0, ix1, ix2, ix3,
               n_acc, d_acc, gsem):
    cid = lax.axis_index("c")
    sid = lax.axis_index("s")
    wid = sid * NC + cid

    pltpu.sync_copy(as_hbm, asrc_v)
    for h in range(HEADS):
        pltpu.sync_copy(ad_hbm.at[pl.ds(h * NSP, ACC_ROWS)],
                        adst_v.at[pl.ds(h * ACC_ROWS, ACC_ROWS)])

    @pl.when(sid == 0)
    def _():
        pltpu.sync_copy(zn_hbm, n_acc)
        pltpu.sync_copy(zd_hbm, d_acc)

    plsc.subcore_barrier()

    lanes = lax.iota(_i32, 16)
    ixs = (ix0, ix1, ix2, ix3)

    def chunk(k, _):
        e0 = wid * EPW + k * CH
        pltpu.sync_copy(src_hbm.at[pl.ds(e0, CH)], sv)
        pltpu.sync_copy(dst_hbm.at[pl.ds(e0, CH)], dv)
        pltpu.sync_copy(w_hbm.at[pl.ds(e0, CH)], wv)
        cp = pltpu.async_copy(xs_hbm.at[sv], G, gsem)

        for j in range(CH // 16):
            svec = sv[pl.ds(j * 16, 16)]
            dvec = dv[pl.ds(j * 16, 16)]
            wvec = wv[pl.ds(j * 16, 16)]
            d4 = dvec * 4
            for h in range(HEADS):
                a = (plsc.load_gather(asrc_v, [svec + (h * NSP)])
                     + plsc.load_gather(adst_v, [dvec + (h * ACC_ROWS)]))
                a = jnp.where(a >= 0.0, a, a * NEG_SLOPE)
                p = jnp.exp(a)
                pH[pl.ds(h * CH + j * 16, 16)] = p
                cT[pl.ds(h * CH + j * 16, 16)] = p * wvec
                ixs[h][pl.ds(j * 16, 16)] = d4 + h

        cp.wait()

        def scale(g, _):
            evec = lanes + g * 16
            for h in range(HEADS):
                cvec = cT[pl.ds(h * CH + g * 16, 16)]
                for i in range(DIM):
                    col = jnp.full((16,), h * DIM + i, dtype=_i32)
                    vals = plsc.load_gather(G, [evec, col])
                    plsc.store_scatter(G, [evec, col], vals * cvec)
            return 0

        lax.fori_loop(0, CH // 16, scale, 0)

        pltpu.sync_copy(G, n_acc.at[dv], add=True)
        for h in range(HEADS):
            pltpu.sync_copy(pH.at[pl.ds(h * CH, CH)], d_acc.at[ixs[h]],
                            add=True)
        return 0

    lax.fori_loop(0, NCH, chunk, 0)
    plsc.subcore_barrier()

    r0 = sid * RPT
    pltpu.sync_copy(n_acc.at[pl.ds(r0, RPT)], np_hbm.at[cid, pl.ds(r0, RPT)])
    pltpu.sync_copy(d_acc.at[pl.ds(r0 * 4, RPT * 4)],
                    dp_hbm.at[cid, pl.ds(r0 * 4, RPT * 4)])


def _sc_edge(src_pad, dst_pad, w_pad, xs_tab, as_flat, ad_flat, zn, zd):
    mesh = plsc.VectorSubcoreMesh(core_axis_name="c", subcore_axis_name="s")
    f = functools.partial(
        pl.kernel,
        mesh=mesh,
        compiler_params=pltpu.CompilerParams(needs_layout_passes=False),
        out_type=[
            jax.ShapeDtypeStruct((NC, ACC_ROWS, HID), _f32),
            jax.ShapeDtypeStruct((NC, ACC_ROWS * HEADS), _f32),
        ],
        scratch_types=[
            pltpu.VMEM((HEADS * NSP,), _f32),
            pltpu.VMEM((HEADS * ACC_ROWS,), _f32),
            pltpu.VMEM((CH,), _i32),
            pltpu.VMEM((CH,), _i32),
            pltpu.VMEM((CH,), _f32),
            pltpu.VMEM((CH, HID), _f32),
            pltpu.VMEM((HEADS * CH,), _f32),
            pltpu.VMEM((HEADS * CH,), _f32),
            pltpu.VMEM((CH,), _i32),
            pltpu.VMEM((CH,), _i32),
            pltpu.VMEM((CH,), _i32),
            pltpu.VMEM((CH,), _i32),
            pltpu.VMEM_SHARED((ACC_ROWS, HID), _f32),
            pltpu.VMEM_SHARED((ACC_ROWS * HEADS,), _f32),
            pltpu.SemaphoreType.DMA,
        ],
    )(_edge_body)
    return f(src_pad, dst_pad, w_pad, xs_tab, as_flat, ad_flat, zn, zd)


_BF = 1000


def _final_body(np_ref, dp_ref, xs_ref, as_ref, ad_ref, gb_ref, out_ref):
    n = np_ref[0] + np_ref[1]
    d = dp_ref[0] + dp_ref[1]
    a = as_ref[...] + ad_ref[...]
    a = jnp.where(a >= 0.0, a, a * NEG_SLOPE)
    p = jnp.exp(a)
    dt = d + p

    def expand(v):
        return jnp.concatenate(
            [jnp.broadcast_to(v[:, h:h + 1], (_BF, DIM)) for h in range(HEADS)],
            axis=1)

    num = n + expand(p) * xs_ref[...]
    out_ref[...] = num / expand(dt) + gb_ref[...]


def _tc_final(np_, dp_nm, xs_tab, as_nm, ad_nm, gb2):
    grid = (N_DST // _BF,)
    return pl.pallas_call(
        _final_body,
        grid=grid,
        in_specs=[
            pl.BlockSpec((NC, _BF, HID), lambda i: (0, i, 0)),
            pl.BlockSpec((NC, _BF, HEADS), lambda i: (0, i, 0)),
            pl.BlockSpec((_BF, HID), lambda i: (i, 0)),
            pl.BlockSpec((_BF, HEADS), lambda i: (i, 0)),
            pl.BlockSpec((_BF, HEADS), lambda i: (i, 0)),
            pl.BlockSpec((1, HID), lambda i: (0, 0)),
        ],
        out_specs=pl.BlockSpec((_BF, HID), lambda i: (i, 0)),
        out_shape=jax.ShapeDtypeStruct((N_DST, HID), _f32),
    )(np_, dp_nm, xs_tab, as_nm, ad_nm, gb2)


def kernel(n_id, edge_src, edge_dst, weights, size0, size1, pre_w, pre_b,
           lin_src_w, lin_dst_w, att_src, att_dst, gat_bias):
    n_id = n_id.astype(_i32)
    edge_src = edge_src.astype(_i32)
    edge_dst = edge_dst.astype(_i32)

    nid_pad = jnp.concatenate([n_id, jnp.zeros((NSP - N_SRC,), _i32)])
    x_colT = _sc_transpose_gather(pre_w, nid_pad)
    xs_tab, a_srcT, a_dstT = _tc_dense(
        x_colT, lin_src_w, lin_dst_w,
        pre_b.reshape(1, HID), att_src.reshape(HID, 1), att_dst.reshape(HID, 1))

    npad = EPAD - E
    src_pad = jnp.concatenate([edge_src, jnp.zeros((npad,), _i32)])
    dst_pad = jnp.concatenate([edge_dst, jnp.full((npad,), N_DST, _i32)])
    w_pad = jnp.concatenate([weights, jnp.zeros((npad,), _f32)])
    zn = jnp.zeros((ACC_ROWS, HID), _f32)
    zd = jnp.zeros((ACC_ROWS * HEADS,), _f32)

    np_, dp_ = _sc_edge(src_pad, dst_pad, w_pad, xs_tab,
                        a_srcT.reshape(-1), a_dstT.reshape(-1), zn, zd)
    return _tc_final(np_, dp_.reshape(NC, ACC_ROWS, HEADS), xs_tab,
                     a_srcT.T, a_dstT.T, gat_bias.reshape(1, HID))


# speedup vs baseline: 18.0017x; 1.0738x over previous
import functools

import jax
import jax.numpy as jnp
from jax import lax
from jax.experimental import pallas as pl
from jax.experimental.pallas import tpu as pltpu
from jax.experimental.pallas import tpu_sc as plsc

IN_SIZE = 100000
DIM = 32
HEADS = 4
HID = DIM * HEADS
N_SRC = 10000
NSP = 10240
N_DST = 5000
E = 320000
NEG_SLOPE = 0.1

NC = 2
NS = 16
NW = NC * NS
CH = 128
NCH = 80
EPW = CH * NCH
EPAD = EPW * NW
ACC_ROWS = 5120
RPT = ACC_ROWS // NS

_f32 = jnp.float32
_i32 = jnp.int32


def _xpose_body(pre_w_hbm, nid_hbm, out_hbm, rowbuf, nid_v, outv):
    cid = lax.axis_index("c")
    sid = lax.axis_index("s")
    wid = sid * NC + cid
    pltpu.sync_copy(nid_hbm, nid_v)
    nrow = HID // NW

    def do_row(q, _):
        r = wid * nrow + q
        pltpu.sync_copy(pre_w_hbm.at[r], rowbuf)

        def gth(j, _):
            idx = nid_v[pl.ds(j * 16, 16)]
            outv[pl.ds(j * 16, 16)] = plsc.load_gather(rowbuf, [idx])
            return 0

        lax.fori_loop(0, NSP // 16, gth, 0)
        pltpu.sync_copy(outv, out_hbm.at[r])
        return 0

    lax.fori_loop(0, nrow, do_row, 0)


def _sc_transpose_gather(pre_w, n_id):
    mesh = plsc.VectorSubcoreMesh(core_axis_name="c", subcore_axis_name="s")
    f = functools.partial(
        pl.kernel,
        mesh=mesh,
        compiler_params=pltpu.CompilerParams(needs_layout_passes=False),
        out_type=jax.ShapeDtypeStruct((HID, NSP), _f32),
        scratch_types=[
            pltpu.VMEM((IN_SIZE,), _f32),
            pltpu.VMEM((NSP,), _i32),
            pltpu.VMEM((NSP,), _f32),
        ],
    )(_xpose_body)
    return f(pre_w, n_id)


_BN = 1024


def _dense_body(xc_ref, ws_ref, wd_ref, pb_ref, ats_ref, atd_ref,
                xs_out, as_out, ad_out):
    xc = xc_ref[...]
    ws = ws_ref[...]
    wd = wd_ref[...]
    pb = pb_ref[...]
    dn = (((0,), (1,)), ((), ()))
    bias_s = lax.dot_general(pb, ws, dimension_numbers=(((1,), (1,)), ((), ())),
                             preferred_element_type=_f32)
    bias_d = lax.dot_general(pb, wd, dimension_numbers=(((1,), (1,)), ((), ())),
                             preferred_element_type=_f32)
    xs = lax.dot_general(xc, ws, dimension_numbers=dn,
                         preferred_element_type=_f32) + bias_s
    xs_out[...] = xs

    ts = ws * ats_ref[...]
    td = wd * atd_ref[...]
    vs = jnp.concatenate(
        [jnp.sum(ts[DIM * h:DIM * (h + 1), :], axis=0, keepdims=True)
         for h in range(HEADS)], axis=0)
    vd = jnp.concatenate(
        [jnp.sum(td[DIM * h:DIM * (h + 1), :], axis=0, keepdims=True)
         for h in range(HEADS)], axis=0)
    bs = lax.dot_general(vs, pb, dimension_numbers=(((1,), (1,)), ((), ())),
                         preferred_element_type=_f32)
    bd = lax.dot_general(vd, pb, dimension_numbers=(((1,), (1,)), ((), ())),
                         preferred_element_type=_f32)
    as_out[...] = lax.dot_general(vs, xc, dimension_numbers=(((1,), (0,)), ((), ())),
                                  preferred_element_type=_f32) + bs
    ad_out[...] = lax.dot_general(vd, xc, dimension_numbers=(((1,), (0,)), ((), ())),
                                  preferred_element_type=_f32) + bd


def _tc_dense(x_colT, lin_src_w, lin_dst_w, pre_b2, ats2, atd2):
    grid = (NSP // _BN,)
    return pl.pallas_call(
        _dense_body,
        grid=grid,
        in_specs=[
            pl.BlockSpec((HID, _BN), lambda i: (0, i)),
            pl.BlockSpec((HID, HID), lambda i: (0, 0)),
            pl.BlockSpec((HID, HID), lambda i: (0, 0)),
            pl.BlockSpec((1, HID), lambda i: (0, 0)),
            pl.BlockSpec((HID, 1), lambda i: (0, 0)),
            pl.BlockSpec((HID, 1), lambda i: (0, 0)),
        ],
        out_specs=[
            pl.BlockSpec((_BN, HID), lambda i: (i, 0)),
            pl.BlockSpec((HEADS, _BN), lambda i: (0, i)),
            pl.BlockSpec((HEADS, _BN), lambda i: (0, i)),
        ],
        out_shape=[
            jax.ShapeDtypeStruct((NSP, HID), _f32),
            jax.ShapeDtypeStruct((HEADS, NSP), _f32),
            jax.ShapeDtypeStruct((HEADS, NSP), _f32),
        ],
    )(x_colT, lin_src_w, lin_dst_w, pre_b2, ats2, atd2)


def _edge_body(edat_hbm, xs_hbm, as_hbm, ad_hbm, zn_hbm, zd_hbm,
               np_hbm, dp_hbm,
               asrc_v, adst_v, ebuf, dv0, dv1, G, M, pH, cT,
               ix0, ix1, ix2, ix3, n_acc, d_acc, gsem):
    cid = lax.axis_index("c")
    sid = lax.axis_index("s")
    wid = sid * NC + cid

    pltpu.sync_copy(as_hbm, asrc_v)
    for h in range(HEADS):
        pltpu.sync_copy(ad_hbm.at[pl.ds(h * NSP, ACC_ROWS)],
                        adst_v.at[pl.ds(h * ACC_ROWS, ACC_ROWS)])

    @pl.when(sid == 0)
    def _():
        pltpu.sync_copy(zn_hbm, n_acc)
        pltpu.sync_copy(zd_hbm, d_acc)

    plsc.subcore_barrier()

    ixs = (ix0, ix1, ix2, ix3)
    lanes = lax.iota(_i32, 16)

    def chunk(k, _):
        pltpu.sync_copy(edat_hbm.at[wid * NCH + k], ebuf)
        cp = pltpu.async_copy(xs_hbm.at[ebuf.at[pl.ds(0, CH)]], G, gsem)

        for j in range(CH // 16):
            svec = ebuf[pl.ds(j * 16, 16)]
            dvec = ebuf[pl.ds(CH + j * 16, 16)]
            wvec = plsc.bitcast(ebuf[pl.ds(2 * CH + j * 16, 16)], _f32)
            d4 = dvec * 4
            for h in range(HEADS):
                a = (plsc.load_gather(asrc_v, [svec + (h * NSP)])
                     + plsc.load_gather(adst_v, [dvec + (h * ACC_ROWS)]))
                a = jnp.where(a >= 0.0, a, a * NEG_SLOPE)
                p = jnp.exp(a)
                pH[pl.ds(h * CH + j * 16, 16)] = p
                cT[pl.ds(h * CH + j * 16, 16)] = p * wvec
                ixs[h][pl.ds(j * 16, 16)] = d4 + h
        for i in range(4):
            dv0[pl.ds(i * 16, 16)] = ebuf[pl.ds(CH + i * 16, 16)]
            dv1[pl.ds(i * 16, 16)] = ebuf[pl.ds(CH + 64 + i * 16, 16)]

        cp.wait()

        def scale_half(base, dvh):
            def sgrp(g, _):
                evec = lanes + (base + g * 16)
                mvec = lanes + g * 16
                for h in range(HEADS):
                    cvec = cT[pl.ds(h * CH + base + g * 16, 16)]
                    for i in range(DIM):
                        col = jnp.full((16,), h * DIM + i, dtype=_i32)
                        vals = plsc.load_gather(G, [evec, col])
                        plsc.store_scatter(M, [mvec, col], vals * cvec)
                return 0

            lax.fori_loop(0, 64 // 16, sgrp, 0)
            pltpu.sync_copy(M, n_acc.at[dvh], add=True)

        scale_half(0, dv0)
        scale_half(64, dv1)

        for h in range(HEADS):
            pltpu.sync_copy(pH.at[pl.ds(h * CH, CH)], d_acc.at[ixs[h]],
                            add=True)
        return 0

    lax.fori_loop(0, NCH, chunk, 0)
    plsc.subcore_barrier()

    r0 = sid * RPT
    pltpu.sync_copy(n_acc.at[pl.ds(r0, RPT)], np_hbm.at[cid, pl.ds(r0, RPT)])
    pltpu.sync_copy(d_acc.at[pl.ds(r0 * 4, RPT * 4)],
                    dp_hbm.at[cid, pl.ds(r0 * 4, RPT * 4)])


def _sc_edge(edat, xs_tab, as_flat, ad_flat, zn, zd):
    mesh = plsc.VectorSubcoreMesh(core_axis_name="c", subcore_axis_name="s")
    f = functools.partial(
        pl.kernel,
        mesh=mesh,
        compiler_params=pltpu.CompilerParams(needs_layout_passes=False),
        out_type=[
            jax.ShapeDtypeStruct((NC, ACC_ROWS, HID), _f32),
            jax.ShapeDtypeStruct((NC, ACC_ROWS * HEADS), _f32),
        ],
        scratch_types=[
            pltpu.VMEM((HEADS * NSP,), _f32),
            pltpu.VMEM((HEADS * ACC_ROWS,), _f32),
            pltpu.VMEM((3 * CH,), _i32),
            pltpu.VMEM((64,), _i32),
            pltpu.VMEM((64,), _i32),
            pltpu.VMEM((CH, HID), _f32),
            pltpu.VMEM((64, HID), _f32),
            pltpu.VMEM((HEADS * CH,), _f32),
            pltpu.VMEM((HEADS * CH,), _f32),
            pltpu.VMEM((CH,), _i32),
            pltpu.VMEM((CH,), _i32),
            pltpu.VMEM((CH,), _i32),
            pltpu.VMEM((CH,), _i32),
            pltpu.VMEM_SHARED((ACC_ROWS, HID), _f32),
            pltpu.VMEM_SHARED((ACC_ROWS * HEADS,), _f32),
            pltpu.SemaphoreType.DMA,
        ],
    )(_edge_body)
    return f(edat, xs_tab, as_flat, ad_flat, zn, zd)


_BF = 1000


def _final_body(np_ref, dp_ref, xs_ref, as_ref, ad_ref, gb_ref, out_ref):
    n = np_ref[0] + np_ref[1]
    d = dp_ref[0] + dp_ref[1]
    a = as_ref[...] + ad_ref[...]
    a = jnp.where(a >= 0.0, a, a * NEG_SLOPE)
    p = jnp.exp(a)
    dt = d + p

    def expand(v):
        return jnp.concatenate(
            [jnp.broadcast_to(v[:, h:h + 1], (_BF, DIM)) for h in range(HEADS)],
            axis=1)

    num = n + expand(p) * xs_ref[...]
    out_ref[...] = num / expand(dt) + gb_ref[...]


def _tc_final(np_, dp_nm, xs_tab, as_nm, ad_nm, gb2):
    grid = (N_DST // _BF,)
    return pl.pallas_call(
        _final_body,
        grid=grid,
        in_specs=[
            pl.BlockSpec((NC, _BF, HID), lambda i: (0, i, 0)),
            pl.BlockSpec((NC, _BF, HEADS), lambda i: (0, i, 0)),
            pl.BlockSpec((_BF, HID), lambda i: (i, 0)),
            pl.BlockSpec((_BF, HEADS), lambda i: (i, 0)),
            pl.BlockSpec((_BF, HEADS), lambda i: (i, 0)),
            pl.BlockSpec((1, HID), lambda i: (0, 0)),
        ],
        out_specs=pl.BlockSpec((_BF, HID), lambda i: (i, 0)),
        out_shape=jax.ShapeDtypeStruct((N_DST, HID), _f32),
    )(np_, dp_nm, xs_tab, as_nm, ad_nm, gb2)


def kernel(n_id, edge_src, edge_dst, weights, size0, size1, pre_w, pre_b,
           lin_src_w, lin_dst_w, att_src, att_dst, gat_bias):
    n_id = n_id.astype(_i32)
    edge_src = edge_src.astype(_i32)
    edge_dst = edge_dst.astype(_i32)

    nid_pad = jnp.concatenate([n_id, jnp.zeros((NSP - N_SRC,), _i32)])
    x_colT = _sc_transpose_gather(pre_w, nid_pad)
    xs_tab, a_srcT, a_dstT = _tc_dense(
        x_colT, lin_src_w, lin_dst_w,
        pre_b.reshape(1, HID), att_src.reshape(HID, 1), att_dst.reshape(HID, 1))

    npad = EPAD - E
    src_pad = jnp.concatenate([edge_src, jnp.zeros((npad,), _i32)])
    dst_pad = jnp.concatenate([edge_dst, jnp.full((npad,), N_DST, _i32)])
    w_pad = jnp.concatenate([weights, jnp.zeros((npad,), _f32)])
    edat = jnp.stack(
        [src_pad.reshape(-1, CH), dst_pad.reshape(-1, CH),
         lax.bitcast_convert_type(w_pad, _i32).reshape(-1, CH)],
        axis=1).reshape(-1, 3 * CH)
    zn = jnp.zeros((ACC_ROWS, HID), _f32)
    zd = jnp.zeros((ACC_ROWS * HEADS,), _f32)

    np_, dp_ = _sc_edge(edat, xs_tab,
                        a_srcT.reshape(-1), a_dstT.reshape(-1), zn, zd)
    return _tc_final(np_, dp_.reshape(NC, ACC_ROWS, HEADS), xs_tab,
                     a_srcT.T, a_dstT.T, gat_bias.reshape(1, HID))


# speedup vs baseline: 19.7839x; 1.0990x over previous
import functools

import jax
import jax.numpy as jnp
from jax import lax
from jax.experimental import pallas as pl
from jax.experimental.pallas import tpu as pltpu
from jax.experimental.pallas import tpu_sc as plsc

IN_SIZE = 100000
DIM = 32
HEADS = 4
HID = DIM * HEADS
N_SRC = 10000
NSP = 10240
N_DST = 5000
E = 320000
NEG_SLOPE = 0.1

NC = 2
NS = 16
NW = NC * NS
CH = 128
NCH = 80
EPW = CH * NCH
EPAD = EPW * NW
ACC_ROWS = 5120
RPT = ACC_ROWS // NS

_f32 = jnp.float32
_i32 = jnp.int32


def _xpose_body(pre_w_hbm, nid_hbm, out_hbm, rowbuf, nid_v, outv):
    cid = lax.axis_index("c")
    sid = lax.axis_index("s")
    wid = sid * NC + cid
    pltpu.sync_copy(nid_hbm, nid_v)
    nrow = HID // NW

    def do_row(q, _):
        r = wid * nrow + q
        pltpu.sync_copy(pre_w_hbm.at[r], rowbuf)

        def gth(j, _):
            idx = nid_v[pl.ds(j * 16, 16)]
            outv[pl.ds(j * 16, 16)] = plsc.load_gather(rowbuf, [idx])
            return 0

        lax.fori_loop(0, NSP // 16, gth, 0)
        pltpu.sync_copy(outv, out_hbm.at[r])
        return 0

    lax.fori_loop(0, nrow, do_row, 0)


def _sc_transpose_gather(pre_w, n_id):
    mesh = plsc.VectorSubcoreMesh(core_axis_name="c", subcore_axis_name="s")
    f = functools.partial(
        pl.kernel,
        mesh=mesh,
        compiler_params=pltpu.CompilerParams(needs_layout_passes=False),
        out_type=jax.ShapeDtypeStruct((HID, NSP), _f32),
        scratch_types=[
            pltpu.VMEM((IN_SIZE,), _f32),
            pltpu.VMEM((NSP,), _i32),
            pltpu.VMEM((NSP,), _f32),
        ],
    )(_xpose_body)
    return f(pre_w, n_id)


_BN = 1024


def _dense_body(xc_ref, ws_ref, wd_ref, pb_ref, ats_ref, atd_ref,
                xs_out, as_out, ad_out):
    xc = xc_ref[...]
    ws = ws_ref[...]
    wd = wd_ref[...]
    pb = pb_ref[...]
    dn = (((0,), (1,)), ((), ()))
    bias_s = lax.dot_general(pb, ws, dimension_numbers=(((1,), (1,)), ((), ())),
                             preferred_element_type=_f32)
    bias_d = lax.dot_general(pb, wd, dimension_numbers=(((1,), (1,)), ((), ())),
                             preferred_element_type=_f32)
    xs_out[...] = lax.dot_general(xc, ws, dimension_numbers=dn,
                                  preferred_element_type=_f32) + bias_s

    def head_fold(w, att_col):
        t = w * att_col
        return jnp.concatenate(
            [jnp.sum(t[DIM * h:DIM * (h + 1), :], axis=0, keepdims=True)
             for h in range(HEADS)], axis=0)

    vs = head_fold(ws, ats_ref[...])
    vd = head_fold(wd, atd_ref[...])
    bs = lax.dot_general(vs, pb, dimension_numbers=(((1,), (1,)), ((), ())),
                         preferred_element_type=_f32)
    bd = lax.dot_general(vd, pb, dimension_numbers=(((1,), (1,)), ((), ())),
                         preferred_element_type=_f32)
    as_out[...] = lax.dot_general(vs, xc, dimension_numbers=(((1,), (0,)), ((), ())),
                                  preferred_element_type=_f32) + bs
    ad_out[...] = lax.dot_general(vd, xc, dimension_numbers=(((1,), (0,)), ((), ())),
                                  preferred_element_type=_f32) + bd


def _tc_dense(x_colT, lin_src_w, lin_dst_w, pre_b2, ats2, atd2):
    grid = (NSP // _BN,)
    return pl.pallas_call(
        _dense_body,
        grid=grid,
        in_specs=[
            pl.BlockSpec((HID, _BN), lambda i: (0, i)),
            pl.BlockSpec((HID, HID), lambda i: (0, 0)),
            pl.BlockSpec((HID, HID), lambda i: (0, 0)),
            pl.BlockSpec((1, HID), lambda i: (0, 0)),
            pl.BlockSpec((HID, 1), lambda i: (0, 0)),
            pl.BlockSpec((HID, 1), lambda i: (0, 0)),
        ],
        out_specs=[
            pl.BlockSpec((_BN, HID), lambda i: (i, 0)),
            pl.BlockSpec((HEADS, _BN), lambda i: (0, i)),
            pl.BlockSpec((HEADS, _BN), lambda i: (0, i)),
        ],
        out_shape=[
            jax.ShapeDtypeStruct((NSP, HID), _f32),
            jax.ShapeDtypeStruct((HEADS, NSP), _f32),
            jax.ShapeDtypeStruct((HEADS, NSP), _f32),
        ],
    )(x_colT, lin_src_w, lin_dst_w, pre_b2, ats2, atd2)


def _edge_body(edat_hbm, xs_hbm, as_hbm, ad_hbm, zn_hbm, zd_hbm,
               np_hbm, dp_hbm,
               adst_v, ebA, ebB, dv0, dv1, Ga, Gb, M, cT, d_local,
               ixa0, ixa1, ixa2, ixa3, asg,
               n_acc, gsemA, gsemB, esem, asem):
    cid = lax.axis_index("c")
    sid = lax.axis_index("s")
    wid = sid * NC + cid
    row0 = wid * NCH

    for h in range(HEADS):
        pltpu.sync_copy(ad_hbm.at[pl.ds(h * NSP, ACC_ROWS)],
                        adst_v.at[pl.ds(h * ACC_ROWS, ACC_ROWS)])
    pltpu.sync_copy(zd_hbm, d_local)

    @pl.when(sid == 0)
    def _():
        pltpu.sync_copy(zn_hbm, n_acc)

    plsc.subcore_barrier()

    lanes = lax.iota(_i32, 16)
    ixas = (ixa0, ixa1, ixa2, ixa3)

    pltpu.sync_copy(edat_hbm.at[row0], ebA)
    pltpu.async_copy(xs_hbm.at[ebA.at[pl.ds(0, CH)]], Ga, gsemA)
    pltpu.async_copy(edat_hbm.at[row0 + 1], ebB, esem)

    def chunkwork(k, ecur, enxt, Gcur, Gnxt, gsem_cur, gsem_nxt):
        @pl.when(k < NCH - 1)
        def _():
            pltpu.make_async_copy(edat_hbm.at[row0], enxt, esem).wait()
            pltpu.async_copy(xs_hbm.at[enxt.at[pl.ds(0, CH)]], Gnxt, gsem_nxt)

        for j in range(CH // 16):
            svec = ecur[pl.ds(j * 16, 16)]
            for h in range(HEADS):
                ixas[h][pl.ds(j * 16, 16)] = svec + (h * NSP)
        for i in range(4):
            dv0[pl.ds(i * 16, 16)] = ecur[pl.ds(CH + i * 16, 16)]
            dv1[pl.ds(i * 16, 16)] = ecur[pl.ds(CH + 64 + i * 16, 16)]
        for h in range(HEADS):
            pltpu.async_copy(as_hbm.at[ixas[h]], asg.at[pl.ds(h * CH, CH)],
                             asem)
        for h in range(HEADS):
            pltpu.make_async_copy(as_hbm.at[ixas[h]],
                                  asg.at[pl.ds(h * CH, CH)], asem).wait()

        for j in range(CH // 16):
            dvec = ecur[pl.ds(CH + j * 16, 16)]
            wvec = plsc.bitcast(ecur[pl.ds(2 * CH + j * 16, 16)], _f32)
            d4 = dvec * 4
            for h in range(HEADS):
                a = (asg[pl.ds(h * CH + j * 16, 16)]
                     + plsc.load_gather(adst_v, [dvec + (h * ACC_ROWS)]))
                a = jnp.where(a >= 0.0, a, a * NEG_SLOPE)
                p = jnp.exp(a)
                cT[pl.ds(h * CH + j * 16, 16)] = p * wvec
                plsc.addupdate_scatter(d_local, [d4 + h], p)

        @pl.when(k < NCH - 2)
        def _():
            pltpu.async_copy(edat_hbm.at[row0 + k + 2], ecur, esem)

        pltpu.make_async_copy(xs_hbm.at[ecur.at[pl.ds(0, CH)]], Gcur,
                              gsem_cur).wait()

        def scale_half(base, dvh):
            def sgrp(g, _):
                evec2 = lanes + (base + g * 16)
                mvec = lanes + g * 16
                for h in range(HEADS):
                    cvec = cT[pl.ds(h * CH + base + g * 16, 16)]
                    for i in range(DIM):
                        col = jnp.full((16,), h * DIM + i, dtype=_i32)
                        vals = plsc.load_gather(Gcur, [evec2, col])
                        plsc.store_scatter(M, [mvec, col], vals * cvec)
                return 0

            lax.fori_loop(0, 64 // 16, sgrp, 0)
            pltpu.sync_copy(M, n_acc.at[dvh], add=True)

        scale_half(0, dv0)
        scale_half(64, dv1)

    def pair(i, _):
        chunkwork(2 * i, ebA, ebB, Ga, Gb, gsemA, gsemB)
        chunkwork(2 * i + 1, ebB, ebA, Gb, Ga, gsemB, gsemA)
        return 0

    lax.fori_loop(0, NCH // 2, pair, 0)
    plsc.subcore_barrier()

    r0 = sid * RPT
    pltpu.sync_copy(n_acc.at[pl.ds(r0, RPT)], np_hbm.at[cid, pl.ds(r0, RPT)])
    pltpu.sync_copy(d_local, dp_hbm.at[wid])


def _sc_edge(edat, xs_tab, as_flat, ad_flat, zn, zd):
    mesh = plsc.VectorSubcoreMesh(core_axis_name="c", subcore_axis_name="s")
    f = functools.partial(
        pl.kernel,
        mesh=mesh,
        compiler_params=pltpu.CompilerParams(needs_layout_passes=False),
        out_type=[
            jax.ShapeDtypeStruct((NC, ACC_ROWS, HID), _f32),
            jax.ShapeDtypeStruct((NW, ACC_ROWS * HEADS), _f32),
        ],
        scratch_types=[
            pltpu.VMEM((HEADS * ACC_ROWS,), _f32),
            pltpu.VMEM((3 * CH,), _i32),
            pltpu.VMEM((3 * CH,), _i32),
            pltpu.VMEM((64,), _i32),
            pltpu.VMEM((64,), _i32),
            pltpu.VMEM((CH, HID), _f32),
            pltpu.VMEM((CH, HID), _f32),
            pltpu.VMEM((64, HID), _f32),
            pltpu.VMEM((HEADS * CH,), _f32),
            pltpu.VMEM((ACC_ROWS * HEADS,), _f32),
            pltpu.VMEM((CH,), _i32),
            pltpu.VMEM((CH,), _i32),
            pltpu.VMEM((CH,), _i32),
            pltpu.VMEM((CH,), _i32),
            pltpu.VMEM((HEADS * CH,), _f32),
            pltpu.VMEM_SHARED((ACC_ROWS, HID), _f32),
            pltpu.SemaphoreType.DMA,
            pltpu.SemaphoreType.DMA,
            pltpu.SemaphoreType.DMA,
            pltpu.SemaphoreType.DMA,
        ],
    )(_edge_body)
    return f(edat, xs_tab, as_flat, ad_flat, zn, zd)


_BF = 1000


def _final_body(np_ref, dp_ref, xs_ref, as_ref, ad_ref, gb_ref, out_ref):
    n = np_ref[0] + np_ref[1]
    d = dp_ref[0]
    for t in range(1, NW):
        d = d + dp_ref[t]
    a = as_ref[...] + ad_ref[...]
    a = jnp.where(a >= 0.0, a, a * NEG_SLOPE)
    p = jnp.exp(a)
    dt = d + p

    def expand(v):
        return jnp.concatenate(
            [jnp.broadcast_to(v[:, h:h + 1], (_BF, DIM)) for h in range(HEADS)],
            axis=1)

    num = n + expand(p) * xs_ref[...]
    out_ref[...] = num / expand(dt) + gb_ref[...]


def _tc_final(np_, dp_nm, xs_tab, as_nm, ad_nm, gb2):
    grid = (N_DST // _BF,)
    return pl.pallas_call(
        _final_body,
        grid=grid,
        in_specs=[
            pl.BlockSpec((NC, _BF, HID), lambda i: (0, i, 0)),
            pl.BlockSpec((NW, _BF, HEADS), lambda i: (0, i, 0)),
            pl.BlockSpec((_BF, HID), lambda i: (i, 0)),
            pl.BlockSpec((_BF, HEADS), lambda i: (i, 0)),
            pl.BlockSpec((_BF, HEADS), lambda i: (i, 0)),
            pl.BlockSpec((1, HID), lambda i: (0, 0)),
        ],
        out_specs=pl.BlockSpec((_BF, HID), lambda i: (i, 0)),
        out_shape=jax.ShapeDtypeStruct((N_DST, HID), _f32),
    )(np_, dp_nm, xs_tab, as_nm, ad_nm, gb2)


def kernel(n_id, edge_src, edge_dst, weights, size0, size1, pre_w, pre_b,
           lin_src_w, lin_dst_w, att_src, att_dst, gat_bias):
    n_id = n_id.astype(_i32)
    edge_src = edge_src.astype(_i32)
    edge_dst = edge_dst.astype(_i32)

    nid_pad = jnp.concatenate([n_id, jnp.zeros((NSP - N_SRC,), _i32)])
    x_colT = _sc_transpose_gather(pre_w, nid_pad)
    xs_tab, a_srcT, a_dstT = _tc_dense(
        x_colT, lin_src_w, lin_dst_w,
        pre_b.reshape(1, HID), att_src.reshape(HID, 1), att_dst.reshape(HID, 1))

    npad = EPAD - E
    src_pad = jnp.concatenate([edge_src, jnp.zeros((npad,), _i32)])
    dst_pad = jnp.concatenate([edge_dst, jnp.full((npad,), N_DST, _i32)])
    w_pad = jnp.concatenate([weights, jnp.zeros((npad,), _f32)])
    edat = jnp.stack(
        [src_pad.reshape(-1, CH), dst_pad.reshape(-1, CH),
         lax.bitcast_convert_type(w_pad, _i32).reshape(-1, CH)],
        axis=1).reshape(-1, 3 * CH)
    zn = jnp.zeros((ACC_ROWS, HID), _f32)
    zd = jnp.zeros((ACC_ROWS * HEADS,), _f32)

    np_, dp_ = _sc_edge(edat, xs_tab, a_srcT.reshape(-1),
                        a_dstT.reshape(-1), zn, zd)
    return _tc_final(np_, dp_.reshape(NW, ACC_ROWS, HEADS), xs_tab,
                     a_srcT.T, a_dstT.T, gat_bias.reshape(1, HID))


# speedup vs baseline: 55.9400x; 2.8276x over previous
import functools

import jax
import jax.numpy as jnp
from jax import lax
from jax.experimental import pallas as pl
from jax.experimental.pallas import tpu as pltpu
from jax.experimental.pallas import tpu_sc as plsc

IN_SIZE = 100000
DIM = 32
HEADS = 4
HID = DIM * HEADS
N_SRC = 10000
NSP = 10240
N_DST = 5000
E = 320000
NEG_SLOPE = 0.1

NC = 2
NS = 16
NW = NC * NS
CH = 128
NCH = 80
EPW = CH * NCH
EPAD = EPW * NW
ACC_ROWS = 5120
RPT = ACC_ROWS // NS

_f32 = jnp.float32
_i32 = jnp.int32


def _xpose_body(pre_w_hbm, nid_hbm, out_hbm, rowbuf, nid_v, outv):
    cid = lax.axis_index("c")
    sid = lax.axis_index("s")
    wid = sid * NC + cid
    pltpu.sync_copy(nid_hbm, nid_v)
    nrow = HID // NW

    def do_row(q, _):
        r = wid * nrow + q
        pltpu.sync_copy(pre_w_hbm.at[r], rowbuf)

        def gth(j, _):
            idx = nid_v[pl.ds(j * 16, 16)]
            outv[pl.ds(j * 16, 16)] = plsc.load_gather(rowbuf, [idx])
            return 0

        lax.fori_loop(0, NSP // 16, gth, 0)
        pltpu.sync_copy(outv, out_hbm.at[r])
        return 0

    lax.fori_loop(0, nrow, do_row, 0)


def _sc_transpose_gather(pre_w, n_id):
    mesh = plsc.VectorSubcoreMesh(core_axis_name="c", subcore_axis_name="s")
    f = functools.partial(
        pl.kernel,
        mesh=mesh,
        compiler_params=pltpu.CompilerParams(needs_layout_passes=False),
        out_type=jax.ShapeDtypeStruct((HID, NSP), _f32),
        scratch_types=[
            pltpu.VMEM((IN_SIZE,), _f32),
            pltpu.VMEM((NSP,), _i32),
            pltpu.VMEM((NSP,), _f32),
        ],
    )(_xpose_body)
    return f(pre_w, n_id)


_BN = 1024


def _dense_body(xc_ref, ws_ref, wd_ref, pb_ref, ats_ref, atd_ref,
                xs_out, as_out, ad_out):
    xc = xc_ref[...]
    ws = ws_ref[...]
    wd = wd_ref[...]
    pb = pb_ref[...]
    dn = (((0,), (1,)), ((), ()))
    bias_s = lax.dot_general(pb, ws, dimension_numbers=(((1,), (1,)), ((), ())),
                             preferred_element_type=_f32)
    bias_d = lax.dot_general(pb, wd, dimension_numbers=(((1,), (1,)), ((), ())),
                             preferred_element_type=_f32)
    xs_out[...] = lax.dot_general(xc, ws, dimension_numbers=dn,
                                  preferred_element_type=_f32) + bias_s

    def head_fold(w, att_col):
        t = w * att_col
        return jnp.concatenate(
            [jnp.sum(t[DIM * h:DIM * (h + 1), :], axis=0, keepdims=True)
             for h in range(HEADS)], axis=0)

    vs = head_fold(ws, ats_ref[...])
    vd = head_fold(wd, atd_ref[...])
    bs = lax.dot_general(vs, pb, dimension_numbers=(((1,), (1,)), ((), ())),
                         preferred_element_type=_f32)
    bd = lax.dot_general(vd, pb, dimension_numbers=(((1,), (1,)), ((), ())),
                         preferred_element_type=_f32)
    as_out[...] = lax.dot_general(vs, xc, dimension_numbers=(((1,), (0,)), ((), ())),
                                  preferred_element_type=_f32) + bs
    ad_out[...] = lax.dot_general(vd, xc, dimension_numbers=(((1,), (0,)), ((), ())),
                                  preferred_element_type=_f32) + bd


def _tc_dense(x_colT, lin_src_w, lin_dst_w, pre_b2, ats2, atd2):
    grid = (NSP // _BN,)
    return pl.pallas_call(
        _dense_body,
        grid=grid,
        in_specs=[
            pl.BlockSpec((HID, _BN), lambda i: (0, i)),
            pl.BlockSpec((HID, HID), lambda i: (0, 0)),
            pl.BlockSpec((HID, HID), lambda i: (0, 0)),
            pl.BlockSpec((1, HID), lambda i: (0, 0)),
            pl.BlockSpec((HID, 1), lambda i: (0, 0)),
            pl.BlockSpec((HID, 1), lambda i: (0, 0)),
        ],
        out_specs=[
            pl.BlockSpec((_BN, HID), lambda i: (i, 0)),
            pl.BlockSpec((HEADS, _BN), lambda i: (0, i)),
            pl.BlockSpec((HEADS, _BN), lambda i: (0, i)),
        ],
        out_shape=[
            jax.ShapeDtypeStruct((NSP, HID), _f32),
            jax.ShapeDtypeStruct((HEADS, NSP), _f32),
            jax.ShapeDtypeStruct((HEADS, NSP), _f32),
        ],
    )(x_colT, lin_src_w, lin_dst_w, pre_b2, ats2, atd2)


def _edge_body(edat_hbm, xs_hbm, as_hbm, ad_hbm, zn_hbm, zd_hbm,
               np_hbm, dp_hbm,
               adst_v, ebA, ebB, dv0, dv1, Ga, Gb, M, cT, d_local,
               ixa0, ixa1, ixa2, ixa3, asg,
               n_acc, gsemA, gsemB, esem, asem):
    cid = lax.axis_index("c")
    sid = lax.axis_index("s")
    wid = sid * NC + cid
    row0 = wid * NCH

    for h in range(HEADS):
        pltpu.sync_copy(ad_hbm.at[pl.ds(h * NSP, ACC_ROWS)],
                        adst_v.at[pl.ds(h * ACC_ROWS, ACC_ROWS)])
    pltpu.sync_copy(zd_hbm, d_local)

    @pl.when(sid == 0)
    def _():
        pltpu.sync_copy(zn_hbm, n_acc)

    plsc.subcore_barrier()

    lanes = lax.iota(_i32, 16)
    ixas = (ixa0, ixa1, ixa2, ixa3)

    pltpu.sync_copy(edat_hbm.at[row0], ebA)
    pltpu.async_copy(xs_hbm.at[ebA.at[pl.ds(0, CH)]], Ga, gsemA)
    pltpu.async_copy(edat_hbm.at[row0 + 1], ebB, esem)

    def chunkwork(k, ecur, enxt, Gcur, Gnxt, gsem_cur, gsem_nxt):
        @pl.when(k < NCH - 1)
        def _():
            pltpu.make_async_copy(edat_hbm.at[row0], enxt, esem).wait()
            pltpu.async_copy(xs_hbm.at[enxt.at[pl.ds(0, CH)]], Gnxt, gsem_nxt)

        for j in range(CH // 16):
            svec = ecur[pl.ds(j * 16, 16)]
            for h in range(HEADS):
                ixas[h][pl.ds(j * 16, 16)] = svec + (h * NSP)
        for i in range(4):
            dv0[pl.ds(i * 16, 16)] = ecur[pl.ds(CH + i * 16, 16)]
            dv1[pl.ds(i * 16, 16)] = ecur[pl.ds(CH + 64 + i * 16, 16)]
        for h in range(HEADS):
            pltpu.async_copy(as_hbm.at[ixas[h]], asg.at[pl.ds(h * CH, CH)],
                             asem)
        for h in range(HEADS):
            pltpu.make_async_copy(as_hbm.at[ixas[h]],
                                  asg.at[pl.ds(h * CH, CH)], asem).wait()

        for j in range(CH // 16):
            dvec = ecur[pl.ds(CH + j * 16, 16)]
            wvec = plsc.bitcast(ecur[pl.ds(2 * CH + j * 16, 16)], _f32)
            d4 = dvec * 4
            for h in range(HEADS):
                a = (asg[pl.ds(h * CH + j * 16, 16)]
                     + plsc.load_gather(adst_v, [dvec + (h * ACC_ROWS)]))
                a = jnp.where(a >= 0.0, a, a * NEG_SLOPE)
                p = jnp.exp(a)
                cT[pl.ds(h * CH + j * 16, 16)] = p * wvec
                plsc.addupdate_scatter(d_local, [d4 + h], p)

        @pl.when(k < NCH - 2)
        def _():
            pltpu.async_copy(edat_hbm.at[row0 + k + 2], ecur, esem)

        pltpu.make_async_copy(xs_hbm.at[ecur.at[pl.ds(0, CH)]], Gcur,
                              gsem_cur).wait()

        def scale_half(base, dvh):
            def sgrp(g, _):
                for h in range(HEADS):
                    cvec = cT[pl.ds(h * CH + base + g * 16, 16)]
                    for l in range(16):
                        row = g * 16 + l
                        cs = jnp.full((16,), cvec[l], dtype=_f32)
                        for i2 in range(DIM // 16):
                            d0 = h * DIM + 16 * i2
                            M[row, pl.ds(d0, 16)] = (
                                Gcur[base + row, pl.ds(d0, 16)] * cs)
                return 0

            lax.fori_loop(0, 64 // 16, sgrp, 0)
            pltpu.sync_copy(M, n_acc.at[dvh], add=True)

        scale_half(0, dv0)
        scale_half(64, dv1)

    def pair(i, _):
        chunkwork(2 * i, ebA, ebB, Ga, Gb, gsemA, gsemB)
        chunkwork(2 * i + 1, ebB, ebA, Gb, Ga, gsemB, gsemA)
        return 0

    lax.fori_loop(0, NCH // 2, pair, 0)
    plsc.subcore_barrier()

    r0 = sid * RPT
    pltpu.sync_copy(n_acc.at[pl.ds(r0, RPT)], np_hbm.at[cid, pl.ds(r0, RPT)])
    pltpu.sync_copy(d_local, dp_hbm.at[wid])


def _sc_edge(edat, xs_tab, as_flat, ad_flat, zn, zd):
    mesh = plsc.VectorSubcoreMesh(core_axis_name="c", subcore_axis_name="s")
    f = functools.partial(
        pl.kernel,
        mesh=mesh,
        compiler_params=pltpu.CompilerParams(needs_layout_passes=False),
        out_type=[
            jax.ShapeDtypeStruct((NC, ACC_ROWS, HID), _f32),
            jax.ShapeDtypeStruct((NW, ACC_ROWS * HEADS), _f32),
        ],
        scratch_types=[
            pltpu.VMEM((HEADS * ACC_ROWS,), _f32),
            pltpu.VMEM((3 * CH,), _i32),
            pltpu.VMEM((3 * CH,), _i32),
            pltpu.VMEM((64,), _i32),
            pltpu.VMEM((64,), _i32),
            pltpu.VMEM((CH, HID), _f32),
            pltpu.VMEM((CH, HID), _f32),
            pltpu.VMEM((64, HID), _f32),
            pltpu.VMEM((HEADS * CH,), _f32),
            pltpu.VMEM((ACC_ROWS * HEADS,), _f32),
            pltpu.VMEM((CH,), _i32),
            pltpu.VMEM((CH,), _i32),
            pltpu.VMEM((CH,), _i32),
            pltpu.VMEM((CH,), _i32),
            pltpu.VMEM((HEADS * CH,), _f32),
            pltpu.VMEM_SHARED((ACC_ROWS, HID), _f32),
            pltpu.SemaphoreType.DMA,
            pltpu.SemaphoreType.DMA,
            pltpu.SemaphoreType.DMA,
            pltpu.SemaphoreType.DMA,
        ],
    )(_edge_body)
    return f(edat, xs_tab, as_flat, ad_flat, zn, zd)


_BF = 1000


def _final_body(np_ref, dp_ref, xs_ref, as_ref, ad_ref, gb_ref, out_ref):
    n = np_ref[0] + np_ref[1]
    d = dp_ref[0]
    for t in range(1, NW):
        d = d + dp_ref[t]
    a = as_ref[...] + ad_ref[...]
    a = jnp.where(a >= 0.0, a, a * NEG_SLOPE)
    p = jnp.exp(a)
    dt = d + p

    def expand(v):
        return jnp.concatenate(
            [jnp.broadcast_to(v[:, h:h + 1], (_BF, DIM)) for h in range(HEADS)],
            axis=1)

    num = n + expand(p) * xs_ref[...]
    out_ref[...] = num / expand(dt) + gb_ref[...]


def _tc_final(np_, dp_nm, xs_tab, as_nm, ad_nm, gb2):
    grid = (N_DST // _BF,)
    return pl.pallas_call(
        _final_body,
        grid=grid,
        in_specs=[
            pl.BlockSpec((NC, _BF, HID), lambda i: (0, i, 0)),
            pl.BlockSpec((NW, _BF, HEADS), lambda i: (0, i, 0)),
            pl.BlockSpec((_BF, HID), lambda i: (i, 0)),
            pl.BlockSpec((_BF, HEADS), lambda i: (i, 0)),
            pl.BlockSpec((_BF, HEADS), lambda i: (i, 0)),
            pl.BlockSpec((1, HID), lambda i: (0, 0)),
        ],
        out_specs=pl.BlockSpec((_BF, HID), lambda i: (i, 0)),
        out_shape=jax.ShapeDtypeStruct((N_DST, HID), _f32),
    )(np_, dp_nm, xs_tab, as_nm, ad_nm, gb2)


def kernel(n_id, edge_src, edge_dst, weights, size0, size1, pre_w, pre_b,
           lin_src_w, lin_dst_w, att_src, att_dst, gat_bias):
    n_id = n_id.astype(_i32)
    edge_src = edge_src.astype(_i32)
    edge_dst = edge_dst.astype(_i32)

    nid_pad = jnp.concatenate([n_id, jnp.zeros((NSP - N_SRC,), _i32)])
    x_colT = _sc_transpose_gather(pre_w, nid_pad)
    xs_tab, a_srcT, a_dstT = _tc_dense(
        x_colT, lin_src_w, lin_dst_w,
        pre_b.reshape(1, HID), att_src.reshape(HID, 1), att_dst.reshape(HID, 1))

    npad = EPAD - E
    src_pad = jnp.concatenate([edge_src, jnp.zeros((npad,), _i32)])
    dst_pad = jnp.concatenate([edge_dst, jnp.full((npad,), N_DST, _i32)])
    w_pad = jnp.concatenate([weights, jnp.zeros((npad,), _f32)])
    edat = jnp.stack(
        [src_pad.reshape(-1, CH), dst_pad.reshape(-1, CH),
         lax.bitcast_convert_type(w_pad, _i32).reshape(-1, CH)],
        axis=1).reshape(-1, 3 * CH)
    zn = jnp.zeros((ACC_ROWS, HID), _f32)
    zd = jnp.zeros((ACC_ROWS * HEADS,), _f32)

    np_, dp_ = _sc_edge(edat, xs_tab, a_srcT.reshape(-1),
                        a_dstT.reshape(-1), zn, zd)
    return _tc_final(np_, dp_.reshape(NW, ACC_ROWS, HEADS), xs_tab,
                     a_srcT.T, a_dstT.T, gat_bias.reshape(1, HID))
